# baseline (speedup 1.0000x reference)
import functools

import jax
import jax.numpy as jnp
from jax import lax
from jax.experimental import pallas as pl
from jax.experimental.pallas import tpu as pltpu
from jax.experimental.pallas import tpu_sc as plsc

N_ATOMS = 6400000
N_MOL = 100000
N_MOL_PAD = 100352
NC, NS = 2, 16
NW = NC * NS
PER_W = N_ATOMS // NW
CH = 8000
ITERS = PER_W // CH
NVEC = CH // 16
U = 4
SEG = 512
FLUSH_AT = 4288
PCAP = FLUSH_AT + CH + 16
NSEG_MAX = (PCAP + SEG - 1) // SEG
ZCH = N_MOL_PAD // NS


def _sc_kernel_body(e1_hbm, e2_hbm, idx_hbm, tot_hbm, part_hbm,
                    e1a, e1b, e2a, e2b, ixa, ixb, tta, ttb,
                    pvals, pidx, dvals, acc_sh,
                    sin_a, sin_b, sout_a, sout_b):
    e1v = (e1a, e1b)
    e2v = (e2a, e2b)
    ixv = (ixa, ixb)
    ttv = (tta, ttb)
    sin = (sin_a, sin_b)
    sout = (sout_a, sout_b)

    cc = lax.axis_index("c")
    ss = lax.axis_index("s")
    base = (cc * NS + ss) * PER_W

    zf = jnp.zeros((16,), jnp.float32)
    zi = jnp.zeros((16,), jnp.int32)

    @plsc.parallel_loop(0, ZCH // 16)
    def _(i):
        e1a[pl.ds(i * 16, 16)] = zf
    pltpu.sync_copy(e1a.at[pl.ds(0, ZCH)], acc_sh.at[pl.ds(ss * ZCH, ZCH)])

    @plsc.parallel_loop(0, PCAP // 16)
    def _(i):
        pidx[pl.ds(i * 16, 16)] = zi
    pvals[pl.ds(0, 16)] = zf
    plsc.subcore_barrier()

    def start_in(j):
        s = j % 2
        off = base + j * CH
        ilen = CH + 16 if j < ITERS - 1 else CH
        return (
            pltpu.async_copy(e1_hbm.at[pl.ds(off, CH)], e1v[s], sin[s]),
            pltpu.async_copy(e2_hbm.at[pl.ds(off, CH)], e2v[s], sin[s]),
            pltpu.async_copy(idx_hbm.at[pl.ds(off, ilen)],
                             ixv[s].at[pl.ds(0, ilen)], sin[s]),
        )

    def flush(cnt, run, pad_idx, force):
        pvals[pl.ds(cnt + 8, 16)] = jnp.full((16,), run, jnp.float32)
        pidx[pl.ds(cnt, 16)] = pad_idx
        do = force | (cnt >= FLUSH_AT)
        tripd = jnp.where(do, (cnt >> 4) + 1, 0)
        nseg = jnp.where(do, (cnt + 16 + (SEG - 1)) >> 9, 0)

        @plsc.parallel_loop(0, tripd)
        def _(k):
            hi = pvals[pl.ds(k * 16 + 8, 16)]
            lo = pvals[pl.ds(k * 16 + 7, 16)]
            dvals[pl.ds(k * 16, 16)] = hi - lo

        @plsc.parallel_loop(0, nseg * 32 - tripd)
        def _(k):
            dvals[pl.ds(tripd * 16 + k * 16, 16)] = zf

        def sbody(k, _):
            pltpu.sync_copy(dvals.at[pl.ds(k * SEG, SEG)],
                            acc_sh.at[pidx.at[pl.ds(k * SEG, SEG)]],
                            add=True)
            return 0
        lax.fori_loop(0, nseg, sbody, 0)

        old = pvals[pl.ds(0, 16)]
        pvals[pl.ds(0, 16)] = jnp.where(do, jnp.full((16,), run, jnp.float32),
                                        old)
        return jnp.where(do, 0, cnt)

    in_d = {0: start_in(0), 1: start_in(1)}
    out_d = {}
    cnt = jnp.int32(0)
    run = jnp.float32(0.0)

    for j in range(ITERS):
        s = j % 2
        for d in in_d.pop(j):
            d.wait()
        if j == ITERS - 1:
            ixv[s][pl.ds(CH, 16)] = jnp.full((16,), -1, jnp.int32)
        if j - 2 in out_d:
            out_d.pop(j - 2).wait()

        def body(i, carry):
            cnt, run = carry
            csl, ms, ids, pcl = [], [], [], []
            for u in range(U):
                o = (i * U + u) * 16
                a = e1v[s][pl.ds(o, 16)]
                b = e2v[s][pl.ds(o, 16)]
                v = a + b
                ttv[s][pl.ds(o, 16)] = v
                idv = ixv[s][pl.ds(o, 16)]
                idn = ixv[s][pl.ds(o + 1, 16)]
                m = idv != idn
                cs = lax.cumsum(v, axis=0)
                csl.append(cs)
                ms.append(m)
                ids.append(idv)
                pcl.append(plsc.all_reduce_population_count(m)[0])
            for u in range(U):
                plsc.store_compressed(pvals.at[pl.ds(cnt + 8, 16)],
                                      csl[u] + run, mask=ms[u])
                plsc.store_compressed(pidx.at[pl.ds(cnt, 16)],
                                      ids[u], mask=ms[u])
                run = run + csl[u][15]
                cnt = cnt + pcl[u]
            return cnt, run

        cnt, run = plsc.parallel_loop(
            0, NVEC // U, carry=(cnt, run))(body)

        out_d[j] = pltpu.async_copy(
            ttv[s], tot_hbm.at[pl.ds(base + j * CH, CH)], sout[s])

        if j < ITERS - 1:
            cnt = flush(cnt, run, ixv[s][pl.ds(CH, 16)], jnp.bool_(False))
        else:
            cnt = flush(cnt, run, zi, jnp.bool_(True))

        if j + 2 < ITERS:
            in_d[j + 2] = start_in(j + 2)

    for j in sorted(out_d):
        out_d.pop(j).wait()

    plsc.subcore_barrier()
    pltpu.sync_copy(acc_sh.at[pl.ds(ss * ZCH, ZCH)],
                    part_hbm.at[cc, pl.ds(ss * ZCH, ZCH)])


_sc_call = functools.partial(
    pl.kernel,
    out_type=[
        jax.ShapeDtypeStruct((N_ATOMS,), jnp.float32),
        jax.ShapeDtypeStruct((NC, N_MOL_PAD), jnp.float32),
    ],
    mesh=plsc.VectorSubcoreMesh(core_axis_name="c", subcore_axis_name="s"),
    compiler_params=pltpu.CompilerParams(needs_layout_passes=False),
    scratch_types=[
        pltpu.VMEM((CH,), jnp.float32),
        pltpu.VMEM((CH,), jnp.float32),
        pltpu.VMEM((CH,), jnp.float32),
        pltpu.VMEM((CH,), jnp.float32),
        pltpu.VMEM((CH + 16,), jnp.int32),
        pltpu.VMEM((CH + 16,), jnp.int32),
        pltpu.VMEM((CH,), jnp.float32),
        pltpu.VMEM((CH,), jnp.float32),
        pltpu.VMEM((8 + PCAP,), jnp.float32),
        pltpu.VMEM((PCAP,), jnp.int32),
        pltpu.VMEM((NSEG_MAX * SEG,), jnp.float32),
        pltpu.VMEM_SHARED((N_MOL_PAD,), jnp.float32),
        pltpu.SemaphoreType.DMA,
        pltpu.SemaphoreType.DMA,
        pltpu.SemaphoreType.DMA,
        pltpu.SemaphoreType.DMA,
    ],
)(_sc_kernel_body)


def _combine_body(parts_ref, out_ref):
    out_ref[...] = parts_ref[0:1, :] + parts_ref[1:2, :]


_combine = pl.pallas_call(
    _combine_body,
    out_shape=jax.ShapeDtypeStruct((1, N_MOL_PAD), jnp.float32),
)


@jax.jit
def kernel(atom_energy_1, atom_energy_2, mol_index, n_molecules):
    e1 = atom_energy_1.reshape(N_ATOMS)
    e2 = atom_energy_2.reshape(N_ATOMS)
    idx = mol_index.astype(jnp.int32)
    total, parts = _sc_call(e1, e2, idx)
    mol = _combine(parts)
    mol_energy = mol.reshape(N_MOL_PAD)[:N_MOL].reshape(N_MOL, 1)
    return (mol_energy, total.reshape(N_ATOMS, 1))


# speedup vs baseline: 1.0040x; 1.0040x over previous
import functools

import jax
import jax.numpy as jnp
from jax import lax
from jax.experimental import pallas as pl
from jax.experimental.pallas import tpu as pltpu
from jax.experimental.pallas import tpu_sc as plsc

N_ATOMS = 6400000
N_MOL = 100000
N_MOL_PAD = 100352
NC, NS = 2, 16
NW = NC * NS
PER_W = N_ATOMS // NW
CH = 8000
ITERS = PER_W // CH
NVEC = CH // 16
U = 5
SEG = 512
FLUSH_AT = 4288
PCAP = FLUSH_AT + CH + 16
NSEG_MAX = (PCAP + SEG - 1) // SEG
ZCH = N_MOL_PAD // NS


def _sc_kernel_body(e1_hbm, e2_hbm, idx_hbm, tot_hbm, part_hbm,
                    e1a, e1b, e2a, e2b, ixa, ixb, tta, ttb,
                    pvals, pidx, dvals, acc_sh,
                    sin_a, sin_b, sout_a, sout_b):
    e1v = (e1a, e1b)
    e2v = (e2a, e2b)
    ixv = (ixa, ixb)
    ttv = (tta, ttb)
    sin = (sin_a, sin_b)
    sout = (sout_a, sout_b)

    cc = lax.axis_index("c")
    ss = lax.axis_index("s")
    base = (cc * NS + ss) * PER_W

    zf = jnp.zeros((16,), jnp.float32)
    zi = jnp.zeros((16,), jnp.int32)

    @plsc.parallel_loop(0, ZCH // 16)
    def _(i):
        e1a[pl.ds(i * 16, 16)] = zf
    pltpu.sync_copy(e1a.at[pl.ds(0, ZCH)], acc_sh.at[pl.ds(ss * ZCH, ZCH)])

    @plsc.parallel_loop(0, PCAP // 16)
    def _(i):
        pidx[pl.ds(i * 16, 16)] = zi
    pvals[pl.ds(0, 16)] = zf
    plsc.subcore_barrier()

    def start_in(j):
        s = j % 2
        off = base + j * CH
        ilen = CH + 16 if j < ITERS - 1 else CH
        return (
            pltpu.async_copy(e1_hbm.at[pl.ds(off, CH)], e1v[s], sin[s]),
            pltpu.async_copy(e2_hbm.at[pl.ds(off, CH)], e2v[s], sin[s]),
            pltpu.async_copy(idx_hbm.at[pl.ds(off, ilen)],
                             ixv[s].at[pl.ds(0, ilen)], sin[s]),
        )

    def flush(cnt, run, pad_idx, force):
        pvals[pl.ds(cnt + 8, 16)] = jnp.full((16,), run, jnp.float32)
        pidx[pl.ds(cnt, 16)] = pad_idx
        do = force | (cnt >= FLUSH_AT)
        tripd = jnp.where(do, (cnt >> 4) + 1, 0)
        nseg = jnp.where(do, (cnt + 16 + (SEG - 1)) >> 9, 0)

        @plsc.parallel_loop(0, tripd)
        def _(k):
            hi = pvals[pl.ds(k * 16 + 8, 16)]
            lo = pvals[pl.ds(k * 16 + 7, 16)]
            dvals[pl.ds(k * 16, 16)] = hi - lo

        @plsc.parallel_loop(0, nseg * 32 - tripd)
        def _(k):
            dvals[pl.ds(tripd * 16 + k * 16, 16)] = zf

        def sbody(k, _):
            pltpu.sync_copy(dvals.at[pl.ds(k * SEG, SEG)],
                            acc_sh.at[pidx.at[pl.ds(k * SEG, SEG)]],
                            add=True)
            return 0
        lax.fori_loop(0, nseg, sbody, 0)

        old = pvals[pl.ds(0, 16)]
        pvals[pl.ds(0, 16)] = jnp.where(do, jnp.full((16,), run, jnp.float32),
                                        old)
        return jnp.where(do, 0, cnt)

    in_d = {0: start_in(0), 1: start_in(1)}
    out_d = {}
    cnt = jnp.int32(0)
    run = jnp.float32(0.0)

    for j in range(ITERS):
        s = j % 2
        for d in in_d.pop(j):
            d.wait()
        if j == ITERS - 1:
            ixv[s][pl.ds(CH, 16)] = jnp.full((16,), -1, jnp.int32)
        if j - 2 in out_d:
            out_d.pop(j - 2).wait()

        def body(i, carry):
            cnt, run = carry
            csl, ms, ids, pcl = [], [], [], []
            for u in range(U):
                o = (i * U + u) * 16
                a = e1v[s][pl.ds(o, 16)]
                b = e2v[s][pl.ds(o, 16)]
                v = a + b
                ttv[s][pl.ds(o, 16)] = v
                idv = ixv[s][pl.ds(o, 16)]
                idn = ixv[s][pl.ds(o + 1, 16)]
                m = idv != idn
                cs = lax.cumsum(v, axis=0)
                csl.append(cs)
                ms.append(m)
                ids.append(idv)
                pcl.append(plsc.all_reduce_population_count(m)[0])
            for u in range(U):
                plsc.store_compressed(pvals.at[pl.ds(cnt + 8, 16)],
                                      csl[u] + run, mask=ms[u])
                plsc.store_compressed(pidx.at[pl.ds(cnt, 16)],
                                      ids[u], mask=ms[u])
                run = run + csl[u][15]
                cnt = cnt + pcl[u]
            return cnt, run

        cnt, run = plsc.parallel_loop(
            0, NVEC // U, carry=(cnt, run))(body)

        out_d[j] = pltpu.async_copy(
            ttv[s], tot_hbm.at[pl.ds(base + j * CH, CH)], sout[s])

        if j < ITERS - 1:
            cnt = flush(cnt, run, ixv[s][pl.ds(CH, 16)], jnp.bool_(False))
        else:
            cnt = flush(cnt, run, zi, jnp.bool_(True))

        if j + 2 < ITERS:
            in_d[j + 2] = start_in(j + 2)

    for j in sorted(out_d):
        out_d.pop(j).wait()

    plsc.subcore_barrier()
    pltpu.sync_copy(acc_sh.at[pl.ds(ss * ZCH, ZCH)],
                    part_hbm.at[cc, pl.ds(ss * ZCH, ZCH)])


_sc_call = functools.partial(
    pl.kernel,
    out_type=[
        jax.ShapeDtypeStruct((N_ATOMS,), jnp.float32),
        jax.ShapeDtypeStruct((NC, N_MOL_PAD), jnp.float32),
    ],
    mesh=plsc.VectorSubcoreMesh(core_axis_name="c", subcore_axis_name="s"),
    compiler_params=pltpu.CompilerParams(needs_layout_passes=False),
    scratch_types=[
        pltpu.VMEM((CH,), jnp.float32),
        pltpu.VMEM((CH,), jnp.float32),
        pltpu.VMEM((CH,), jnp.float32),
        pltpu.VMEM((CH,), jnp.float32),
        pltpu.VMEM((CH + 16,), jnp.int32),
        pltpu.VMEM((CH + 16,), jnp.int32),
        pltpu.VMEM((CH,), jnp.float32),
        pltpu.VMEM((CH,), jnp.float32),
        pltpu.VMEM((8 + PCAP,), jnp.float32),
        pltpu.VMEM((PCAP,), jnp.int32),
        pltpu.VMEM((NSEG_MAX * SEG,), jnp.float32),
        pltpu.VMEM_SHARED((N_MOL_PAD,), jnp.float32),
        pltpu.SemaphoreType.DMA,
        pltpu.SemaphoreType.DMA,
        pltpu.SemaphoreType.DMA,
        pltpu.SemaphoreType.DMA,
    ],
)(_sc_kernel_body)


def _combine_body(parts_ref, out_ref):
    out_ref[...] = parts_ref[0:1, :] + parts_ref[1:2, :]


_combine = pl.pallas_call(
    _combine_body,
    out_shape=jax.ShapeDtypeStruct((1, N_MOL_PAD), jnp.float32),
)


@jax.jit
def kernel(atom_energy_1, atom_energy_2, mol_index, n_molecules):
    e1 = atom_energy_1.reshape(N_ATOMS)
    e2 = atom_energy_2.reshape(N_ATOMS)
    idx = mol_index.astype(jnp.int32)
    total, parts = _sc_call(e1, e2, idx)
    mol = _combine(parts)
    mol_energy = mol.reshape(N_MOL_PAD)[:N_MOL].reshape(N_MOL, 1)
    return (mol_energy, total.reshape(N_ATOMS, 1))
